# baseline (speedup 1.0000x reference)
import functools

import jax
import jax.numpy as jnp
from jax import lax
from jax.experimental import pallas as pl
from jax.experimental.pallas import tpu as pltpu
from jax.experimental.pallas import tpu_sc as plsc

_NC = 2
_NS = 16
_NW = _NC * _NS

_CHUNK = 104


def _second_order_sc(xf128, xsub, emb128, *, B, F, D):
    spw = B // _NW
    rows_pw = spw * F
    nchunks_pw = rows_pw // _CHUNK
    gs = 8
    gsz = gs * F
    cpg = gsz // _CHUNK
    groups = spw // gs
    h = D // 2

    mesh = plsc.VectorSubcoreMesh(core_axis_name="c", subcore_axis_name="s")

    @functools.partial(
        pl.kernel,
        out_type=jax.ShapeDtypeStruct((B,), jnp.float32),
        mesh=mesh,
        compiler_params=pltpu.CompilerParams(
            needs_layout_passes=False, use_tc_tiling_on_sc=True),
        scratch_types=[
            pltpu.VMEM((rows_pw,), jnp.int32),
            pltpu.VMEM((rows_pw + 32,), jnp.int32),
            pltpu.VMEM((gsz, 128), jnp.float32),
            pltpu.VMEM((gsz, 128), jnp.float32),
            pltpu.VMEM((spw + 16,), jnp.float32),
            pltpu.SemaphoreType.DMA,
            pltpu.SemaphoreType.DMA,
        ],
    )
    def k1(xf_hbm, xsub_hbm, emb_hbm, sec_hbm,
           idx_v, sub_v, buf_a, buf_b, sec_v, sem_a, sem_b):
        wid = lax.axis_index("s") * _NC + lax.axis_index("c")
        base = wid * rows_pw
        pltpu.sync_copy(xf_hbm.at[pl.ds(base, rows_pw)], idx_v)
        pltpu.sync_copy(xsub_hbm.at[pl.ds(base, rows_pw)],
                        sub_v.at[pl.ds(0, rows_pw)])

        iota16 = lax.iota(jnp.int32, 16)

        def fire(g, buf, sem):
            for q in range(cpg):
                pltpu.async_copy(
                    emb128_idx(g, q), buf.at[pl.ds(q * _CHUNK, _CHUNK)], sem)

        def emb128_idx(g, q):
            return emb_hbm.at[idx_v.at[pl.ds((g * cpg + q) * _CHUNK, _CHUNK)]]

        def drain(buf, sem):
            for q in range(cpg):
                pltpu.make_async_copy(
                    emb_hbm.at[pl.ds(0, _CHUNK)],
                    buf.at[pl.ds(q * _CHUNK, _CHUNK)], sem).wait()

        def compute(g, buf):
            gbase = g * gsz
            sec_acc = jnp.zeros((16,), jnp.float32)
            for l in range(gs):
                p0 = l * F
                subv0 = sub_v[pl.ds(gbase + p0, 16)]
                subv1 = sub_v[pl.ds(gbase + p0 + 16, 16)]
                off = subv0[0] * D
                s0 = buf[p0, pl.ds(off, 16)]
                s1 = buf[p0, pl.ds(off + h, 16)]
                q0 = s0 * s0
                q1 = s1 * s1
                for f in range(1, F):
                    p = p0 + f
                    sub = subv0[f] if f < 16 else subv1[f - 16]
                    off = sub * D
                    e0 = buf[p, pl.ds(off, 16)]
                    e1 = buf[p, pl.ds(off + h, 16)]
                    s0 = s0 + e0
                    s1 = s1 + e1
                    q0 = q0 + e0 * e0
                    q1 = q1 + e1 * e1
                u = s0 * s0 + s1 * s1 - q0 - q1
                sec = 0.5 * jnp.sum(u)
                sec_acc = jnp.where(iota16 == l, sec, sec_acc)
            sec_v[pl.ds(g * gs, 16)] = sec_acc

        fire(0, buf_a, sem_a)
        def loop_body(k, carry):
            ga = 2 * k
            gb = 2 * k + 1
            fire(gb, buf_b, sem_b)
            drain(buf_a, sem_a)
            compute(ga, buf_a)
            fire(jnp.minimum(ga + 2, groups - 2), buf_a, sem_a)
            drain(buf_b, sem_b)
            compute(gb, buf_b)
            return carry
        lax.fori_loop(0, groups // 2, loop_body, 0)
        drain(buf_a, sem_a)
        pltpu.sync_copy(sec_v.at[pl.ds(0, spw)],
                        sec_hbm.at[pl.ds(wid * spw, spw)])

    return k1(xf128, xsub, emb128)


def _first_order_sc(xfl, lin_flat, bias_vec, sec, *, B, F):
    spw = B // _NW
    rows_pw = spw * F
    nchunks_pw = rows_pw // _CHUNK
    gsz = 16 * F
    cpg = gsz // _CHUNK
    groups = spw // 16

    mesh = plsc.VectorSubcoreMesh(core_axis_name="c", subcore_axis_name="s")

    @functools.partial(
        pl.kernel,
        out_type=jax.ShapeDtypeStruct((B,), jnp.float32),
        mesh=mesh,
        compiler_params=pltpu.CompilerParams(
            needs_layout_passes=False, use_tc_tiling_on_sc=False),
        scratch_types=[
            pltpu.VMEM((rows_pw,), jnp.int32),
            pltpu.VMEM((gsz,), jnp.float32),
            pltpu.VMEM((spw,), jnp.float32),
            pltpu.VMEM((spw,), jnp.float32),
            pltpu.VMEM((16,), jnp.float32),
            pltpu.SemaphoreType.DMA,
        ],
    )
    def k2(xfl_hbm, lin_hbm, bias_hbm, sec_hbm, out_hbm,
           idx_v, lin_v, sec_v, out_v, bias_v, sem):
        wid = lax.axis_index("s") * _NC + lax.axis_index("c")
        base = wid * rows_pw
        pltpu.sync_copy(xfl_hbm.at[pl.ds(base, rows_pw)], idx_v)
        pltpu.sync_copy(sec_hbm.at[pl.ds(wid * spw, spw)], sec_v)
        pltpu.sync_copy(bias_hbm, bias_v)

        def group_body(g, carry):
            cps = []
            for q in range(cpg):
                cps.append(pltpu.async_copy(
                    lin_hbm.at[idx_v.at[pl.ds((g * cpg + q) * _CHUNK, _CHUNK)]],
                    lin_v.at[pl.ds(q * _CHUNK, _CHUNK)], sem))
            for cp in cps:
                cp.wait()

            fo = lin_v[pl.ds(0, 16)]
            for f in range(1, F):
                fo = fo + lin_v[pl.ds(f * 16, 16)]

            z = bias_v[...] + fo + sec_v[pl.ds(g * 16, 16)]
            y = 1.0 / (1.0 + jnp.exp(-z))
            out_v[pl.ds(g * 16, 16)] = y
            return carry

        lax.fori_loop(0, groups, group_body, 0)
        pltpu.sync_copy(out_v, out_hbm.at[pl.ds(wid * spw, spw)])

    return k2(xfl, lin_flat, bias_vec, sec)


def kernel(x, emb_tables, lin_tables, bias):
    B, F = x.shape
    _, V, D = emb_tables.shape
    assert B % (16 * _NW) == 0
    assert (16 * F) % _CHUNK == 0 and V % 4 == 0 and (128 % (4 * D)) == 0

    emb128 = emb_tables.reshape((F * V * D) // 128, 128)
    lin_flat = lin_tables.reshape(F * V)
    offs = (jnp.arange(F, dtype=jnp.int32) * V)[None, :]
    x_off = x + offs
    flat = x_off.reshape(-1)
    xf128 = flat >> 2
    xsub = flat & 3
    xfl = x_off.reshape(B // 16, 16, F).transpose(0, 2, 1).reshape(-1)
    bias_vec = jnp.broadcast_to(bias.astype(jnp.float32), (16,))

    sec = _second_order_sc(xf128, xsub, emb128, B=B, F=F, D=D)
    out = _first_order_sc(xfl, lin_flat, bias_vec, sec, B=B, F=F)
    return out.reshape(B, 1)
